# Initial kernel scaffold; baseline (speedup 1.0000x reference)
import jax
import jax.numpy as jnp
from jax.experimental import pallas as pl


def kernel(x, w0, b0, wres, bres, wlin, blin):
    raise NotImplementedError("write your pallas kernel here")


# speedup vs baseline: 2.4873x; 2.4873x over previous
import functools

import jax
import jax.numpy as jnp
from jax import lax
from jax.experimental import pallas as pl
from jax.experimental.pallas import tpu as pltpu

_DEPTH = 4
_WIDTH = 16
_CPAD = 8
_H = 32
_W = 32
_HW = _H * _W
_NOUT = 10
_NOUT_PAD = 16
_BT = 16


def _fused_kernel(x_ref, w0_ref, b0_ref, wres_ref, bres_ref, wlin_ref,
                  blin_ref, msk_ref, o_ref, *, bt):
    lanes = bt * _HW
    f32 = jnp.float32
    bf16 = jnp.bfloat16

    m_colL = msk_ref[0:1]
    m_colR = msk_ref[1:2]
    m_rowU = msk_ref[2:3]
    m_rowD = msk_ref[3:4]

    def conv3x3(act, w_bf, b):
        a0 = act.astype(bf16)
        ap = pltpu.roll(act, 1, axis=1).astype(bf16) * m_colL
        am = pltpu.roll(act, lanes - 1, axis=1).astype(bf16) * m_colR
        cs = jnp.concatenate([ap, a0, am], axis=0)
        cs_i = pltpu.bitcast(cs, jnp.int32)
        dn = pltpu.bitcast(pltpu.roll(cs_i, _W, axis=1), bf16) * m_rowU
        up = pltpu.bitcast(pltpu.roll(cs_i, lanes - _W, axis=1), bf16) * m_rowD
        stack = jnp.concatenate([dn, cs, up], axis=0)
        return jnp.dot(w_bf, stack, preferred_element_type=f32) + b

    act = conv3x3(x_ref[0], w0_ref[...], b0_ref[...])
    for i in range(_DEPTH):
        y = conv3x3(act, wres_ref[i], bres_ref[i])
        act = jnp.maximum(y, 0.0) + act

    wl = wlin_ref[...]
    rows = []
    for b in range(bt):
        ab = act[:, b * _HW:(b + 1) * _HW].astype(bf16)
        full = jnp.dot(ab, wl, preferred_element_type=f32)
        r = full[0:1, 0:_NOUT_PAD]
        for c in range(1, _WIDTH):
            r = r + full[c:c + 1, c * _NOUT_PAD:(c + 1) * _NOUT_PAD]
        rows.append(r)
    o_ref[0] = jnp.concatenate(rows, axis=0) + blin_ref[...]


def kernel(x, w0, b0, wres, bres, wlin, blin):
    N, Cin, H, W = x.shape
    bt = _BT
    n_pad = pl.cdiv(N, bt) * bt
    G = n_pad // bt
    lanes = bt * _HW

    x = x.reshape(N, Cin, _HW)
    if _CPAD > Cin:
        x = jnp.pad(x, ((0, 0), (0, _CPAD - Cin), (0, 0)))
    if n_pad > N:
        x = jnp.pad(x, ((0, n_pad - N), (0, 0), (0, 0)))
    x = x.reshape(G, bt, _CPAD, _HW).transpose(0, 2, 1, 3).reshape(G, _CPAD, lanes)

    lane = jnp.arange(lanes, dtype=jnp.int32)
    p = lane % _HW
    row = p // _W
    col = p % _W
    msk = jnp.stack([col >= 1, col <= _W - 2, row >= 1, row <= _H - 2]
                    ).astype(jnp.bfloat16)

    w0_bf = w0.astype(jnp.bfloat16)
    wres_bf = wres.astype(jnp.bfloat16)
    wlin_bf = wlin.astype(jnp.bfloat16)

    kern = functools.partial(_fused_kernel, bt=bt)
    const = pl.Buffered(1)
    out = pl.pallas_call(
        kern,
        out_shape=jax.ShapeDtypeStruct((G, bt, _NOUT_PAD), jnp.float32),
        grid_spec=pltpu.PrefetchScalarGridSpec(
            num_scalar_prefetch=0,
            grid=(G,),
            in_specs=[
                pl.BlockSpec((1, _CPAD, lanes), lambda g: (g, 0, 0)),
                pl.BlockSpec((_WIDTH, 9 * _CPAD), lambda g: (0, 0),
                             pipeline_mode=const),
                pl.BlockSpec((_WIDTH, 1), lambda g: (0, 0), pipeline_mode=const),
                pl.BlockSpec((_DEPTH, _WIDTH, 9 * _WIDTH), lambda g: (0, 0, 0),
                             pipeline_mode=const),
                pl.BlockSpec((_DEPTH, _WIDTH, 1), lambda g: (0, 0, 0),
                             pipeline_mode=const),
                pl.BlockSpec((_HW, _WIDTH * _NOUT_PAD), lambda g: (0, 0),
                             pipeline_mode=const),
                pl.BlockSpec((1, _NOUT_PAD), lambda g: (0, 0),
                             pipeline_mode=const),
                pl.BlockSpec((4, lanes), lambda g: (0, 0), pipeline_mode=const),
            ],
            out_specs=pl.BlockSpec((1, bt, _NOUT_PAD), lambda g: (g, 0, 0)),
        ),
        compiler_params=pltpu.CompilerParams(
            dimension_semantics=("parallel",),
            vmem_limit_bytes=48 * 1024 * 1024,
        ),
    )(x, w0_bf, b0, wres_bf, bres, wlin_bf, blin, msk)
    return out.reshape(n_pad, _NOUT_PAD)[:N, :_NOUT]
